# Initial kernel scaffold; baseline (speedup 1.0000x reference)
import jax
import jax.numpy as jnp
from jax.experimental import pallas as pl


def kernel(tokens):
    raise NotImplementedError("write your pallas kernel here")


# speedup vs baseline: 11.1500x; 11.1500x over previous
import functools

import jax
import jax.numpy as jnp
from jax import lax
from jax.experimental import pallas as pl
from jax.experimental.pallas import tpu as pltpu
from jax.experimental.pallas import tpu_sc as plsc

_B, _T = 16384, 4096
_HEAD = 16
_NW = 32
_RPW = _B // _NW
_GR = 256


def _sc_head_swap_body(tokens_hbm, head_hbm, buf):
    wid = lax.axis_index("s") * 2 + lax.axis_index("c")
    base = wid * _RPW
    rows = pl.ds(base, _RPW)
    pltpu.sync_copy(tokens_hbm.at[rows, pl.ds(0, _HEAD)], buf)

    iot = lax.iota(jnp.int32, _HEAD)
    perm = jnp.where(iot == 0, 1, jnp.where(iot == 1, 0, iot))
    dnums = lax.GatherDimensionNumbers(
        offset_dims=(), collapsed_slice_dims=(0,), start_index_map=(0,))

    def step(r, carry):
        v = buf[r, :]
        buf[r, :] = lax.gather(
            v, perm[:, None], dimension_numbers=dnums, slice_sizes=(1,),
            mode=lax.GatherScatterMode.PROMISE_IN_BOUNDS)
        return carry

    lax.fori_loop(0, _RPW, step, 0)
    pltpu.sync_copy(buf, head_hbm.at[rows, pl.ds(0, _HEAD)])


@functools.cache
def _sc_head_swap():
    return pl.kernel(
        _sc_head_swap_body,
        out_type=jax.ShapeDtypeStruct((_B, _HEAD), jnp.float32),
        mesh=plsc.VectorSubcoreMesh(core_axis_name="c", subcore_axis_name="s"),
        scratch_types=[pltpu.VMEM((_RPW, _HEAD), jnp.float32)],
        compiler_params=pltpu.CompilerParams(use_tc_tiling_on_sc=False),
    )


def _tc_copy_body(tok_ref, head_ref, out_ref):
    out_ref[...] = tok_ref[...]
    out_ref[:, 0:_HEAD] = head_ref[...]


@functools.cache
def _tc_copy():
    return pl.pallas_call(
        _tc_copy_body,
        grid=(_B // _GR,),
        in_specs=[
            pl.BlockSpec((_GR, _T), lambda i: (i, 0)),
            pl.BlockSpec((_GR, _HEAD), lambda i: (i, 0)),
        ],
        out_specs=pl.BlockSpec((_GR, _T), lambda i: (i, 0)),
        out_shape=jax.ShapeDtypeStruct((_B, _T), jnp.float32),
        compiler_params=pltpu.CompilerParams(
            dimension_semantics=("arbitrary",),
        ),
    )


def kernel(tokens):
    head = _sc_head_swap()(tokens)
    return _tc_copy()(tokens, head)


# speedup vs baseline: 21.7842x; 1.9537x over previous
import functools

import jax
import jax.numpy as jnp
from jax import lax
from jax.experimental import pallas as pl
from jax.experimental.pallas import tpu as pltpu
from jax.experimental.pallas import tpu_sc as plsc

_B, _T = 16384, 4096
_HEAD = 128
_SWAPW = 16
_NW = 32
_RPW = _B // _NW
_GR = 256


def _sc_head_swap_body(tokens_hbm, head_hbm, buf):
    wid = lax.axis_index("s") * 2 + lax.axis_index("c")
    base = wid * _RPW
    rows = pl.ds(base, _RPW)
    pltpu.sync_copy(tokens_hbm.at[rows, pl.ds(0, _HEAD)], buf)

    iot = lax.iota(jnp.int32, _SWAPW)
    perm = jnp.where(iot == 0, 1, jnp.where(iot == 1, 0, iot))
    dnums = lax.GatherDimensionNumbers(
        offset_dims=(), collapsed_slice_dims=(0,), start_index_map=(0,))

    def step(r, carry):
        v = buf[r, pl.ds(0, _SWAPW)]
        buf[r, pl.ds(0, _SWAPW)] = lax.gather(
            v, perm[:, None], dimension_numbers=dnums, slice_sizes=(1,),
            mode=lax.GatherScatterMode.PROMISE_IN_BOUNDS)
        return carry

    lax.fori_loop(0, _RPW, step, 0)
    pltpu.sync_copy(buf, head_hbm.at[rows, pl.ds(0, _HEAD)])


@functools.cache
def _sc_head_swap():
    return pl.kernel(
        _sc_head_swap_body,
        out_type=jax.ShapeDtypeStruct((_B, _HEAD), jnp.float32),
        mesh=plsc.VectorSubcoreMesh(core_axis_name="c", subcore_axis_name="s"),
        scratch_types=[pltpu.VMEM((_RPW, _HEAD), jnp.float32)],
    )


def _tc_copy_body(tok_ref, head_ref, out_ref):
    out_ref[...] = tok_ref[...]
    out_ref[:, 0:_HEAD] = head_ref[...]


@functools.cache
def _tc_copy():
    return pl.pallas_call(
        _tc_copy_body,
        grid=(_B // _GR,),
        in_specs=[
            pl.BlockSpec((_GR, _T), lambda i: (i, 0)),
            pl.BlockSpec((_GR, _HEAD), lambda i: (i, 0)),
        ],
        out_specs=pl.BlockSpec((_GR, _T), lambda i: (i, 0)),
        out_shape=jax.ShapeDtypeStruct((_B, _T), jnp.float32),
        compiler_params=pltpu.CompilerParams(
            dimension_semantics=("arbitrary",),
        ),
    )


def kernel(tokens):
    head = _sc_head_swap()(tokens)
    return _tc_copy()(tokens, head)


# speedup vs baseline: 21.8976x; 1.0052x over previous
import functools

import jax
import jax.numpy as jnp
from jax import lax
from jax.experimental import pallas as pl
from jax.experimental.pallas import tpu as pltpu
from jax.experimental.pallas import tpu_sc as plsc

_B, _T = 16384, 4096
_HEAD = 128
_SWAPW = 16
_NW = 32
_RPW = _B // _NW
_GR = 512


def _sc_head_swap_body(tokens_hbm, head_hbm, buf):
    wid = lax.axis_index("s") * 2 + lax.axis_index("c")
    base = wid * _RPW
    rows = pl.ds(base, _RPW)
    pltpu.sync_copy(tokens_hbm.at[rows, pl.ds(0, _HEAD)], buf)

    iot = lax.iota(jnp.int32, _SWAPW)
    perm = jnp.where(iot == 0, 1, jnp.where(iot == 1, 0, iot))
    dnums = lax.GatherDimensionNumbers(
        offset_dims=(), collapsed_slice_dims=(0,), start_index_map=(0,))

    def step(r, carry):
        v = buf[r, pl.ds(0, _SWAPW)]
        buf[r, pl.ds(0, _SWAPW)] = lax.gather(
            v, perm[:, None], dimension_numbers=dnums, slice_sizes=(1,),
            mode=lax.GatherScatterMode.PROMISE_IN_BOUNDS)
        return carry

    lax.fori_loop(0, _RPW, step, 0)
    pltpu.sync_copy(buf, head_hbm.at[rows, pl.ds(0, _HEAD)])


@functools.cache
def _sc_head_swap():
    return pl.kernel(
        _sc_head_swap_body,
        out_type=jax.ShapeDtypeStruct((_B, _HEAD), jnp.float32),
        mesh=plsc.VectorSubcoreMesh(core_axis_name="c", subcore_axis_name="s"),
        scratch_types=[pltpu.VMEM((_RPW, _HEAD), jnp.float32)],
    )


def _tc_copy_body(tok_ref, head_ref, out_ref):
    out_ref[...] = tok_ref[...]
    out_ref[:, 0:_HEAD] = head_ref[...]


@functools.cache
def _tc_copy():
    return pl.pallas_call(
        _tc_copy_body,
        grid=(_B // _GR,),
        in_specs=[
            pl.BlockSpec((_GR, _T), lambda i: (i, 0)),
            pl.BlockSpec((_GR, _HEAD), lambda i: (i, 0)),
        ],
        out_specs=pl.BlockSpec((_GR, _T), lambda i: (i, 0)),
        out_shape=jax.ShapeDtypeStruct((_B, _T), jnp.float32),
        compiler_params=pltpu.CompilerParams(
            dimension_semantics=("arbitrary",),
        ),
    )


def kernel(tokens):
    head = _sc_head_swap()(tokens)
    return _tc_copy()(tokens, head)
